# baseline (speedup 1.0000x reference)
import functools

import jax
import jax.numpy as jnp
import numpy as np
from jax import lax
from jax.experimental import pallas as pl
from jax.experimental.pallas import tpu as pltpu

IN_F = 4096
OUT_F = 4096
N_EXP = 8
PACK = 8
N_TOK = 32
BLK_O = 512
QCOL = IN_F // PACK

_PCOL = np.arange(IN_F)
_PROW = (PACK * (_PCOL % QCOL) + _PCOL // QCOL)
_PERM = np.zeros((IN_F, IN_F), dtype=np.float32)
_PERM[_PROW, _PCOL] = 1.0
_PERM_BF16 = jnp.asarray(_PERM, dtype=jnp.bfloat16)


def _unpack_bf16(q):
    parts = [q & 15]
    for n in range(1, PACK - 1):
        parts.append((q >> (4 * n)) & 15)
    parts.append(q >> (4 * (PACK - 1)))
    return jnp.concatenate(parts, axis=1).astype(jnp.bfloat16)


def _body(x_ref, xp_ref, idx_ref, w_ref, q_ref, rs_ref, z_ref, s_ref, o_ref):
    wb = w_ref[...].astype(jnp.bfloat16)
    acc = lax.dot_general(
        x_ref[...], wb, (((1,), (1,)), ((), ())),
        preferred_element_type=jnp.float32,
    )
    for e in range(N_EXP):
        u = _unpack_bf16(q_ref[e])
        xm = jnp.where(idx_ref[...] == e, xp_ref[...], jnp.bfloat16(0))
        dot = lax.dot_general(
            xm, u, (((1,), (1,)), ((), ())),
            preferred_element_type=jnp.float32,
        )
        acc += (dot - rs_ref[0, :, e:e + 1] * z_ref[e]) * s_ref[e]
    o_ref[...] = acc


@jax.jit
def _run(x, xp, idx, weight, qweight, rs, z, s):
    grid = (OUT_F // BLK_O,)
    return pl.pallas_call(
        _body,
        grid=grid,
        in_specs=[
            pl.BlockSpec((N_TOK, IN_F), lambda o: (0, 0)),
            pl.BlockSpec((N_TOK, IN_F), lambda o: (0, 0)),
            pl.BlockSpec((N_TOK, 1), lambda o: (0, 0)),
            pl.BlockSpec((BLK_O, IN_F), lambda o: (o, 0)),
            pl.BlockSpec((N_EXP, BLK_O, QCOL), lambda o: (0, o, 0)),
            pl.BlockSpec((1, N_TOK, N_EXP), lambda o: (0, 0, 0)),
            pl.BlockSpec((N_EXP, 1, BLK_O), lambda o: (0, 0, o)),
            pl.BlockSpec((N_EXP, 1, BLK_O), lambda o: (0, 0, o)),
        ],
        out_specs=pl.BlockSpec((N_TOK, BLK_O), lambda o: (0, o)),
        out_shape=jax.ShapeDtypeStruct((N_TOK, OUT_F), jnp.float32),
        compiler_params=pltpu.CompilerParams(
            dimension_semantics=("arbitrary",),
        ),
    )(x, xp, idx, weight, qweight, rs, z, s)


def kernel(input_, weight, scales_stacked, qweight_stacked, qzeros_stacked, indices):
    x = input_.astype(jnp.bfloat16)
    xp = jnp.dot(
        x, _PERM_BF16, preferred_element_type=jnp.float32
    ).astype(jnp.bfloat16)
    idx = indices.reshape(N_TOK, 1)
    qz = qzeros_stacked.reshape(N_EXP, OUT_F // PACK)
    shifts = jnp.arange(PACK, dtype=jnp.int32) * 4
    z = ((qz[:, :, None] >> shifts) & 15).astype(jnp.float32).reshape(
        N_EXP, 1, OUT_F
    )
    s = scales_stacked.reshape(N_EXP, 1, OUT_F)
    xsum = jnp.sum(xp.astype(jnp.float32), axis=1)
    onehot = indices[:, None] == jnp.arange(N_EXP, dtype=jnp.int32)[None, :]
    rs = (onehot * xsum[:, None]).astype(jnp.float32).reshape(1, N_TOK, N_EXP)
    return _run(x, xp, idx, weight, qweight_stacked, rs, z, s)


# speedup vs baseline: 1.0958x; 1.0958x over previous
import functools

import jax
import jax.numpy as jnp
from jax import lax
from jax.experimental import pallas as pl
from jax.experimental.pallas import tpu as pltpu

IN_F = 4096
OUT_F = 4096
N_EXP = 8
PACK = 8
N_TOK = 32
BLK_O = 512
QCOL = IN_F // PACK


def _unpack_bf16(q):
    parts = [q & 15]
    for n in range(1, PACK - 1):
        parts.append((q >> (4 * n)) & 15)
    parts.append(q >> (4 * (PACK - 1)))
    return jnp.concatenate(parts, axis=1).astype(jnp.bfloat16)


def _body(x_ref, xp_ref, idx_ref, w_ref, q_ref, rs_ref, z_ref, s_ref, o_ref):
    wb = w_ref[...].astype(jnp.bfloat16)
    acc = lax.dot_general(
        x_ref[...], wb, (((1,), (1,)), ((), ())),
        preferred_element_type=jnp.float32,
    )
    for e in range(N_EXP):
        u = _unpack_bf16(q_ref[e])
        xm = jnp.where(idx_ref[...] == e, xp_ref[...], jnp.bfloat16(0))
        dot = lax.dot_general(
            xm, u, (((1,), (1,)), ((), ())),
            preferred_element_type=jnp.float32,
        )
        acc += (dot - rs_ref[0, :, e:e + 1] * z_ref[e]) * s_ref[e]
    o_ref[...] = acc


@jax.jit
def _run(x, xp, idx, weight, qweight, rs, z, s):
    grid = (OUT_F // BLK_O,)
    return pl.pallas_call(
        _body,
        grid=grid,
        in_specs=[
            pl.BlockSpec((N_TOK, IN_F), lambda o: (0, 0)),
            pl.BlockSpec((N_TOK, IN_F), lambda o: (0, 0)),
            pl.BlockSpec((N_TOK, 1), lambda o: (0, 0)),
            pl.BlockSpec((BLK_O, IN_F), lambda o: (o, 0)),
            pl.BlockSpec((N_EXP, BLK_O, QCOL), lambda o: (0, o, 0)),
            pl.BlockSpec((1, N_TOK, N_EXP), lambda o: (0, 0, 0)),
            pl.BlockSpec((N_EXP, 1, BLK_O), lambda o: (0, 0, o)),
            pl.BlockSpec((N_EXP, 1, BLK_O), lambda o: (0, 0, o)),
        ],
        out_specs=pl.BlockSpec((N_TOK, BLK_O), lambda o: (0, o)),
        out_shape=jax.ShapeDtypeStruct((N_TOK, OUT_F), jnp.float32),
        compiler_params=pltpu.CompilerParams(
            dimension_semantics=("arbitrary",),
            vmem_limit_bytes=64 * 1024 * 1024,
        ),
    )(x, xp, idx, weight, qweight, rs, z, s)


def kernel(input_, weight, scales_stacked, qweight_stacked, qzeros_stacked, indices):
    x = input_.astype(jnp.bfloat16)
    xp = (
        input_.reshape(N_TOK, QCOL, PACK)
        .transpose(0, 2, 1)
        .reshape(N_TOK, IN_F)
        .astype(jnp.bfloat16)
    )
    idx = indices.reshape(N_TOK, 1)
    qz = qzeros_stacked.reshape(N_EXP, OUT_F // PACK)
    shifts = jnp.arange(PACK, dtype=jnp.int32) * 4
    z = ((qz[:, :, None] >> shifts) & 15).astype(jnp.float32).reshape(
        N_EXP, 1, OUT_F
    )
    s = scales_stacked.reshape(N_EXP, 1, OUT_F)
    xsum = jnp.sum(xp.astype(jnp.float32), axis=1)
    onehot = indices[:, None] == jnp.arange(N_EXP, dtype=jnp.int32)[None, :]
    rs = (onehot * xsum[:, None]).astype(jnp.float32).reshape(1, N_TOK, N_EXP)
    return _run(x, xp, idx, weight, qweight_stacked, rs, z, s)
